# Initial kernel scaffold; baseline (speedup 1.0000x reference)
import jax
import jax.numpy as jnp
from jax.experimental import pallas as pl


def kernel(edge_index, v_feat, t_feat, user_w, item_w, Wv, bv, Wt, bt, conv_w, conv_b, img_rows, img_cols, img_vals, txt_rows, txt_cols, txt_vals):
    raise NotImplementedError("write your pallas kernel here")


# speedup vs baseline: 5.0082x; 5.0082x over previous
import functools

import jax
import jax.numpy as jnp
from jax import lax
from jax.experimental import pallas as pl
from jax.experimental.pallas import tpu as pltpu
from jax.experimental.pallas import tpu_sc as plsc

N_USER = 10000
N_ITEM = 10000
N_TOT = 20000
D = 64
E = 160000
EK = 100000

NC, NS = 2, 16
NW = NC * NS
BLK = 128
KE = (2 * E + NW * BLK - 1) // (NW * BLK)
EPAD = NW * KE * BLK
KN = (4 * EK + NW * BLK - 1) // (NW * BLK)
NPAD = NW * KN * BLK
ACC_ROWS = 20480
RPS = ACC_ROWS // NS
TRASH = N_TOT
DW = 16

FBLK = 8000
FGB = E // FBLK

_mesh = plsc.VectorSubcoreMesh(core_axis_name="c", subcore_axis_name="s")


def _sc_gather(table, idx, K):
    @functools.partial(
        pl.kernel,
        out_type=jax.ShapeDtypeStruct((NW * K * BLK, D), jnp.float32),
        mesh=_mesh,
        compiler_params=pltpu.CompilerParams(use_tc_tiling_on_sc=False),
        scratch_types=[
            pltpu.VMEM((K, BLK), jnp.int32),
            pltpu.VMEM((BLK, D), jnp.float32),
            pltpu.SemaphoreType.DMA,
        ],
    )
    def k(table_hbm, idx_hbm, out_hbm, idx_v, rows_v, sem):
        cid = lax.axis_index("c")
        sid = lax.axis_index("s")
        wid = sid * NC + cid
        pltpu.sync_copy(idx_hbm.at[wid], idx_v)

        def blk(b, carry):
            pltpu.async_copy(table_hbm.at[idx_v.at[b]], rows_v, sem).wait()
            pltpu.sync_copy(rows_v, out_hbm.at[pl.ds((wid * K + b) * BLK, BLK)])
            return carry

        lax.fori_loop(0, K, blk, 0)

    return k(table, idx)


def _sc_agg(y, src_idx, dst_idx, zeros_acc):
    @functools.partial(
        pl.kernel,
        out_type=jax.ShapeDtypeStruct((NC, ACC_ROWS, D), jnp.float32),
        mesh=_mesh,
        compiler_params=pltpu.CompilerParams(use_tc_tiling_on_sc=False),
        scratch_types=[
            pltpu.VMEM((KE, BLK), jnp.int32),
            pltpu.VMEM((KE, BLK), jnp.int32),
            pltpu.VMEM((BLK, D), jnp.float32),
            pltpu.VMEM_SHARED((ACC_ROWS, D), jnp.float32),
            pltpu.SemaphoreType.DMA,
        ],
    )
    def k(y_hbm, src_hbm, dst_hbm, z_hbm, out_hbm, src_v, dst_v, rows_v, acc_sh, sem):
        cid = lax.axis_index("c")
        sid = lax.axis_index("s")
        wid = sid * NC + cid
        r0 = sid * RPS
        pltpu.sync_copy(z_hbm.at[pl.ds(r0, RPS)], acc_sh.at[pl.ds(r0, RPS)])
        pltpu.sync_copy(src_hbm.at[wid], src_v)
        pltpu.sync_copy(dst_hbm.at[wid], dst_v)
        plsc.subcore_barrier()

        def blk(b, carry):
            pltpu.async_copy(y_hbm.at[src_v.at[b]], rows_v, sem).wait()
            pltpu.sync_copy(rows_v, acc_sh.at[dst_v.at[b]], add=True)
            return carry

        lax.fori_loop(0, KE, blk, 0)
        plsc.subcore_barrier()
        pltpu.sync_copy(acc_sh.at[pl.ds(r0, RPS)], out_hbm.at[cid, pl.ds(r0, RPS)])

    return k(y, src_idx, dst_idx, zeros_acc)


def _sc_deg(mvals, dst_idx, zeros_deg):
    @functools.partial(
        pl.kernel,
        out_type=jax.ShapeDtypeStruct((NC, ACC_ROWS, DW), jnp.float32),
        mesh=_mesh,
        compiler_params=pltpu.CompilerParams(use_tc_tiling_on_sc=False),
        scratch_types=[
            pltpu.VMEM((BLK, DW), jnp.float32),
            pltpu.VMEM((KE, BLK), jnp.int32),
            pltpu.VMEM_SHARED((ACC_ROWS, DW), jnp.float32),
        ],
    )
    def k(m_hbm, dst_hbm, z_hbm, out_hbm, m_v, dst_v, acc_sh):
        cid = lax.axis_index("c")
        sid = lax.axis_index("s")
        wid = sid * NC + cid
        r0 = sid * RPS
        pltpu.sync_copy(z_hbm.at[pl.ds(r0, RPS)], acc_sh.at[pl.ds(r0, RPS)])
        pltpu.sync_copy(dst_hbm.at[wid], dst_v)
        plsc.subcore_barrier()

        def blk(b, carry):
            pltpu.sync_copy(m_hbm.at[wid, b], m_v)
            pltpu.sync_copy(m_v, acc_sh.at[dst_v.at[b]], add=True)
            return carry

        lax.fori_loop(0, KE, blk, 0)
        plsc.subcore_barrier()
        pltpu.sync_copy(acc_sh.at[pl.ds(r0, RPS)], out_hbm.at[cid, pl.ds(r0, RPS)])

    return k(mvals, dst_idx, zeros_deg)


_DOT = dict(preferred_element_type=jnp.float32)


def _mm_bias(x, W, b, blk):
    M, K = x.shape
    Do = W.shape[0]

    def body(x_ref, w_ref, b_ref, o_ref):
        o_ref[...] = lax.dot_general(
            x_ref[...], w_ref[...], (((1,), (1,)), ((), ())), **_DOT
        ) + b_ref[...]

    return pl.pallas_call(
        body,
        grid=(M // blk,),
        in_specs=[
            pl.BlockSpec((blk, K), lambda i: (i, 0)),
            pl.BlockSpec((Do, K), lambda i: (0, 0)),
            pl.BlockSpec((1, Do), lambda i: (0, 0)),
        ],
        out_specs=pl.BlockSpec((blk, Do), lambda i: (i, 0)),
        out_shape=jax.ShapeDtypeStruct((M, Do), jnp.float32),
    )(x, W, b)


def _seg_mix(G1, G2, G3, G4, iv, tv):
    blk = 400

    def body(g1, g2, g3, g4, iv_ref, tv_ref, h_ref, h1_ref, h2_ref):
        ivv = iv_ref[...][:, :, None]
        tvv = tv_ref[...][:, :, None]

        def seg(g, w):
            return jnp.sum(g[...].reshape(blk, 10, D) * w, axis=1)

        def l2(t):
            return t / (jnp.sqrt(jnp.sum(t * t, axis=1, keepdims=True)) + 1e-12)

        t0 = 0.1 * seg(g1, ivv) + 0.9 * seg(g2, tvv)
        h_ref[...] = l2(t0)
        h1_ref[...] = l2(seg(g3, ivv))
        h2_ref[...] = l2(seg(g4, tvv))

    rows = pl.BlockSpec((blk * 10, D), lambda i: (i, 0))
    wspec = pl.BlockSpec((blk, 10), lambda i: (i, 0))
    ospec = pl.BlockSpec((blk, D), lambda i: (i, 0))
    oshape = jax.ShapeDtypeStruct((N_ITEM, D), jnp.float32)
    return pl.pallas_call(
        body,
        grid=(N_ITEM // blk,),
        in_specs=[rows, rows, rows, rows, wspec, wspec],
        out_specs=(ospec, ospec, ospec),
        out_shape=(oshape, oshape, oshape),
    )(G1, G2, G3, G4, iv, tv)


def _filter(a, b, eu3, ev3):
    def body(a_ref, b_ref, eu_ref, ev_ref, m_ref, d1_ref, d2_ref):
        s = jnp.sum(a_ref[...] * b_ref[...], axis=1)
        keep = (s > 0.0).reshape(1, 1, FBLK)
        m_ref[...] = keep.astype(jnp.float32)
        d1_ref[...] = jnp.where(keep, ev_ref[...], TRASH)
        d2_ref[...] = jnp.where(keep, eu_ref[...], TRASH)

    rspec = pl.BlockSpec((FBLK, D), lambda i: (i, 0))
    ispec = pl.BlockSpec((1, 1, FBLK), lambda i: (i, 0, 0))
    ishape = jax.ShapeDtypeStruct((FGB, 1, FBLK), jnp.int32)
    return pl.pallas_call(
        body,
        grid=(FGB,),
        in_specs=[rspec, rspec, ispec, ispec],
        out_specs=(ispec, ispec, ispec),
        out_shape=(jax.ShapeDtypeStruct((FGB, 1, FBLK), jnp.float32), ishape, ishape),
    )(a, b, eu3, ev3)


def _prep_y(x, W, d0, d1):
    blk = 2000

    def body(x_ref, w_ref, d0_ref, d1_ref, y_ref, xw_ref, dis_ref):
        xw = lax.dot_general(x_ref[...], w_ref[...], (((1,), (1,)), ((), ())), **_DOT)
        dis = lax.rsqrt(d0_ref[...] + d1_ref[...] + 1.0)
        xw_ref[...] = xw
        y_ref[...] = dis * xw
        dis_ref[...] = dis

    rspec = pl.BlockSpec((blk, D), lambda i: (i, 0))
    dspec = pl.BlockSpec((blk, 1), lambda i: (i, 0))
    return pl.pallas_call(
        body,
        grid=(N_TOT // blk,),
        in_specs=[
            rspec,
            pl.BlockSpec((D, D), lambda i: (0, 0)),
            dspec,
            dspec,
        ],
        out_specs=(rspec, rspec, dspec),
        out_shape=(
            jax.ShapeDtypeStruct((N_TOT, D), jnp.float32),
            jax.ShapeDtypeStruct((N_TOT, D), jnp.float32),
            jax.ShapeDtypeStruct((N_TOT, 1), jnp.float32),
        ),
    )(x, W, d0, d1)


def _finalize_mid(a0, a1, dis, xw, b, run_prev):
    blk = 2000

    def body(a0_ref, a1_ref, dis_ref, xw_ref, b_ref, rp_ref, x_ref, rn_ref):
        dis = dis_ref[...]
        xn = dis * (a0_ref[...] + a1_ref[...]) + (dis * dis) * xw_ref[...] + b_ref[...]
        x_ref[...] = xn
        rn_ref[...] = rp_ref[...] + xn

    rspec = pl.BlockSpec((blk, D), lambda i: (i, 0))
    dspec = pl.BlockSpec((blk, 1), lambda i: (i, 0))
    oshape = jax.ShapeDtypeStruct((N_TOT, D), jnp.float32)
    return pl.pallas_call(
        body,
        grid=(N_TOT // blk,),
        in_specs=[rspec, rspec, dspec, rspec, pl.BlockSpec((1, D), lambda i: (0, 0)), rspec],
        out_specs=(rspec, rspec),
        out_shape=(oshape, oshape),
    )(a0, a1, dis, xw, b, run_prev)


def _finalize_last(a0, a1, dis, xw, b, run_prev, h_pad):
    blk = 2000

    def body(a0_ref, a1_ref, dis_ref, xw_ref, b_ref, rp_ref, h_ref, o_ref):
        dis = dis_ref[...]
        xn = dis * (a0_ref[...] + a1_ref[...]) + (dis * dis) * xw_ref[...] + b_ref[...]
        o_ref[...] = (rp_ref[...] + xn) * (1.0 / 3.0) + h_ref[...]

    rspec = pl.BlockSpec((blk, D), lambda i: (i, 0))
    dspec = pl.BlockSpec((blk, 1), lambda i: (i, 0))
    return pl.pallas_call(
        body,
        grid=(N_TOT // blk,),
        in_specs=[rspec, rspec, dspec, rspec, pl.BlockSpec((1, D), lambda i: (0, 0)), rspec, rspec],
        out_specs=rspec,
        out_shape=jax.ShapeDtypeStruct((N_TOT, D), jnp.float32),
    )(a0, a1, dis, xw, b, run_prev, h_pad)


def kernel(edge_index, v_feat, t_feat, user_w, item_w, Wv, bv, Wt, bt,
           conv_w, conv_b, img_rows, img_cols, img_vals, txt_rows, txt_cols, txt_vals):
    f32 = jnp.float32
    eu = edge_index[:, 0].astype(jnp.int32)
    ev = edge_index[:, 1].astype(jnp.int32)
    pad_e = EPAD - 2 * E

    src_pad = jnp.concatenate([eu, ev, jnp.zeros((pad_e,), jnp.int32)]).reshape(NW, KE, BLK)
    dst_pad = jnp.concatenate([ev, eu, jnp.full((pad_e,), TRASH, jnp.int32)]).reshape(NW, KE, BLK)
    ones_flat = jnp.concatenate([jnp.ones((2 * E,), f32), jnp.zeros((pad_e,), f32)])
    ones_m = jnp.pad(ones_flat[:, None], ((0, 0), (0, DW - 1))).reshape(NW, KE, BLK, DW)
    knn_idx = jnp.concatenate([
        img_cols, txt_cols, img_cols + N_ITEM, txt_cols + 2 * N_ITEM,
        jnp.zeros((NPAD - 4 * EK,), jnp.int32),
    ]).reshape(NW, KN, BLK)

    zeros_acc = jnp.zeros((ACC_ROWS, D), f32)
    zeros_deg = jnp.zeros((ACC_ROWS, DW), f32)

    v_emb = _mm_bias(v_feat, Wv, bv.reshape(1, -1), 2000)
    t_emb = _mm_bias(t_feat, Wt, bt.reshape(1, -1), 2000)

    Tcat = jnp.concatenate([item_w, v_emb, t_emb], axis=0)
    g = _sc_gather(Tcat, knn_idx, KN)
    iv = img_vals.reshape(N_ITEM, 10)
    tv = txt_vals.reshape(N_ITEM, 10)
    h, h1, h2 = _seg_mix(g[0:EK], g[EK:2 * EK], g[2 * EK:3 * EK], g[3 * EK:4 * EK], iv, tv)

    degF = _sc_deg(ones_m, dst_pad, zeros_deg)
    dF0, dF1 = degF[0, :N_TOT, 0:1], degF[1, :N_TOT, 0:1]

    hz = jnp.zeros((N_USER, D), f32)
    h_pad_v = jnp.concatenate([hz, h1], axis=0)
    h_pad_t = jnp.concatenate([hz, h2], axis=0)
    h_pad_g = jnp.concatenate([hz, h], axis=0)

    eu3 = eu.reshape(FGB, 1, FBLK)
    ev3 = ev.reshape(FGB, 1, FBLK)
    trash_pad_i = jnp.full((pad_e,), TRASH, jnp.int32)
    zeros_pad_f = jnp.zeros((pad_e,), f32)

    def prop(ego, filtered, h_pad):
        x = ego
        run = ego
        for l in range(2):
            W = conv_w[l]
            b = conv_b[l].reshape(1, D)
            if filtered:
                fg = _sc_gather(x, src_pad, KE)
                m3, d13, d23 = _filter(fg[:E], fg[E:2 * E], eu3, ev3)
                m = m3.reshape(E)
                mdst = jnp.concatenate([
                    d13.reshape(E), d23.reshape(E), trash_pad_i,
                ]).reshape(NW, KE, BLK)
                mflat = jnp.concatenate([m, m, zeros_pad_f])
                mvals = jnp.pad(mflat[:, None], ((0, 0), (0, DW - 1))).reshape(NW, KE, BLK, DW)
                degp = _sc_deg(mvals, dst_pad, zeros_deg)
                d0, d1 = degp[0, :N_TOT, 0:1], degp[1, :N_TOT, 0:1]
            else:
                mdst = dst_pad
                d0, d1 = dF0, dF1
            y, xw, dis = _prep_y(x, W, d0, d1)
            agg = _sc_agg(y, src_pad, mdst, zeros_acc)
            a0, a1 = agg[0, :N_TOT], agg[1, :N_TOT]
            if l == 0:
                x, run = _finalize_mid(a0, a1, dis, xw, b, run)
            else:
                return _finalize_last(a0, a1, dis, xw, b, run, h_pad)

    P_v = prop(jnp.concatenate([user_w, v_emb], axis=0), True, h_pad_v)
    P_t = prop(jnp.concatenate([user_w, t_emb], axis=0), True, h_pad_t)
    P_g = prop(jnp.concatenate([user_w, item_w], axis=0), False, h_pad_g)
    return jnp.concatenate([P_g, P_v, P_t], axis=1)
